# baseline (speedup 1.0000x reference)
import functools

import jax
import jax.numpy as jnp
from jax import lax
from jax.experimental import pallas as pl
from jax.experimental.pallas import tpu as pltpu
from jax.experimental.pallas import tpu_sc as plsc

NC, NS, L = 2, 16, 16
NW = NC * NS
B = 16384
D = 64
BPW = B // NW
CHUNK = 128
NCHUNK = BPW // CHUNK


@functools.partial(
    pl.kernel,
    out_type=jax.ShapeDtypeStruct((B,), jnp.float32),
    mesh=plsc.VectorSubcoreMesh(core_axis_name="c", subcore_axis_name="s"),
    scratch_types=[
        pltpu.VMEM((BPW,), jnp.int32),
        pltpu.VMEM((BPW,), jnp.int32),
        pltpu.VMEM((BPW,), jnp.int32),
        pltpu.VMEM((CHUNK, D), jnp.float32),
        pltpu.VMEM((CHUNK, D), jnp.float32),
        pltpu.VMEM((CHUNK, D), jnp.float32),
        pltpu.VMEM((BPW,), jnp.float32),
        pltpu.SemaphoreType.DMA,
    ],
    compiler_params=pltpu.CompilerParams(
        needs_layout_passes=False, use_tc_tiling_on_sc=False),
)
def _transe_sc(trip_hbm, ent_hbm, rel_hbm, out_hbm,
               hidx_v, ridx_v, tidx_v, hbuf, rbuf, tbuf, out_v, sem):
    wid = lax.axis_index("s") * NC + lax.axis_index("c")
    base = wid * BPW
    pltpu.sync_copy(trip_hbm.at[pl.ds(0 * B + base, BPW)], hidx_v)
    pltpu.sync_copy(trip_hbm.at[pl.ds(1 * B + base, BPW)], ridx_v)
    pltpu.sync_copy(trip_hbm.at[pl.ds(2 * B + base, BPW)], tidx_v)

    for c in range(NCHUNK):
        cps = [
            pltpu.async_copy(ent_hbm.at[hidx_v.at[pl.ds(c * CHUNK, CHUNK)]], hbuf, sem),
            pltpu.async_copy(rel_hbm.at[ridx_v.at[pl.ds(c * CHUNK, CHUNK)]], rbuf, sem),
            pltpu.async_copy(ent_hbm.at[tidx_v.at[pl.ds(c * CHUNK, CHUNK)]], tbuf, sem),
        ]
        for cp in cps:
            cp.wait()

        for g in range(CHUNK // L):
            rows = g * L + lax.iota(jnp.int32, L)

            def body(j, acc):
                cols = jnp.full((L,), j, dtype=jnp.int32)
                h = plsc.load_gather(hbuf, [rows, cols])
                r = plsc.load_gather(rbuf, [rows, cols])
                t = plsc.load_gather(tbuf, [rows, cols])
                return acc + jnp.abs(h + r - t)

            dist = lax.fori_loop(0, D, body, jnp.zeros((L,), jnp.float32))
            out_v[pl.ds(c * CHUNK + g * L, L)] = 1.0 / (1.0 + jnp.exp(-dist))

    pltpu.sync_copy(out_v, out_hbm.at[pl.ds(base, BPW)])


def kernel(triplets, ent_embedding, rel_embedding):
    return _transe_sc(triplets.reshape(3 * B), ent_embedding, rel_embedding)


# speedup vs baseline: 1.6022x; 1.6022x over previous
import functools

import jax
import jax.numpy as jnp
from jax import lax
from jax.experimental import pallas as pl
from jax.experimental.pallas import tpu as pltpu
from jax.experimental.pallas import tpu_sc as plsc

NC, NS, L = 2, 16, 16
NW = NC * NS
B = 16384
D = 64
F = 3 * B
NTILE = 7813
TPW = 245
TAIL0 = 7812 * 128
LCAP = 4096
NCH = F // L
CP = pltpu.CompilerParams(needs_layout_passes=False, use_tc_tiling_on_sc=True)
MESH = plsc.VectorSubcoreMesh(core_axis_name="c", subcore_axis_name="s")

_i32 = jnp.int32


def _iota():
    return lax.iota(_i32, L)


@functools.partial(
    pl.kernel,
    out_type=jax.ShapeDtypeStruct((F, D), jnp.float32),
    mesh=MESH,
    scratch_types=[
        pltpu.VMEM((F,), _i32),
        pltpu.VMEM((LCAP + 16,), _i32),
        pltpu.VMEM((LCAP + 16,), _i32),
        pltpu.VMEM((512,), _i32),
        pltpu.VMEM((512,), _i32),
        pltpu.VMEM((128, 128), jnp.float32),
        pltpu.VMEM((128, D), jnp.float32),
        pltpu.VMEM((8, D), jnp.float32),
        pltpu.SMEM((520,), _i32),
        pltpu.SemaphoreType.DMA,
        pltpu.SemaphoreType.DMA,
    ],
    compiler_params=CP,
)
def _gather_sc(trip_hbm, entT, relT, entTailT, relTailT, g_hbm,
               idx_v, lstA, lstB, gkey_v, gcnt_v, tbuf, res_v, trash_v,
               hist_s, sem_t, sem_o):
    wid = lax.axis_index("s") * NC + lax.axis_index("c")
    g0 = wid * TPW
    ghi = jnp.minimum(g0 + TPW, NTILE)
    lane = _iota()
    lane0 = lane == 0
    pltpu.sync_copy(trip_hbm, idx_v)

    def scalar_read(ref, i):
        return plsc.load_gather(ref, [jnp.full((L,), i, dtype=_i32)])[0]

    def scalar_write(ref, i, val):
        plsc.store_scatter(ref, [jnp.full((L,), i, dtype=_i32)],
                           jnp.full((L,), val, dtype=_i32), mask=lane0)

    def round_body(state):
        c_resume, _ = state

        def scan_body(s):
            c, ptr = s
            e = idx_v[pl.ds(c * L, L)]
            tile = lax.shift_right_logical(e, 7)
            m = (tile >= g0) & (tile < ghi)
            is_rel = (lax.shift_right_logical(c, 10) == 1).astype(_i32)
            key = (tile - g0) | (is_rel << 8)
            packed = ((key << 23) | ((e & 127) << 16) | (c * L + lane))
            plsc.store_compressed(lstA.at[pl.ds(ptr, L)], packed, mask=m)
            nm = plsc.all_reduce_population_count(m)[0]
            return c + 1, ptr + nm

        def scan_cond(s):
            c, ptr = s
            return (c < NCH) & (ptr <= LCAP - L)

        c_next, n = lax.while_loop(scan_cond, scan_body, (c_resume, 0))
        lstA[pl.ds(n, L)] = jnp.full((L,), -1, dtype=_i32)
        nch = lax.shift_right_logical(n + L - 1, 4)

        for p in range(9):
            src = lstA if p % 2 == 0 else lstB
            dst = lstB if p % 2 == 0 else lstA
            bit = 23 + p

            def cnt_body(i, z):
                v = src[pl.ds(i * L, L)]
                if bit == 31:
                    m1 = v < 0
                else:
                    m1 = (v & (1 << bit)) != 0
                return z + plsc.all_reduce_population_count(~m1)[0]

            z = lax.fori_loop(0, nch, cnt_body, 0)

            def place_body(i, s):
                p0, p1 = s
                v = src[pl.ds(i * L, L)]
                if bit == 31:
                    m1 = v < 0
                else:
                    m1 = (v & (1 << bit)) != 0
                m0 = ~m1
                plsc.store_compressed(dst.at[pl.ds(p0, L)], v, mask=m0)
                plsc.store_compressed(dst.at[pl.ds(p1, L)], v, mask=m1)
                n0 = plsc.all_reduce_population_count(m0)[0]
                n1 = plsc.all_reduce_population_count(m1)[0]
                return p0 + n0, p1 + n1

            lax.fori_loop(0, nch, place_body, (0, z))

        lst = lstB

        def hz_body(i, _):
            hist_s[i] = 0
            return 0

        lax.fori_loop(0, 512, hz_body, 0)

        def hist_body(i, _):
            v = lst[pl.ds(i * L, L)]
            gidx = i * L + lane
            valid = jnp.where(gidx < n, 1, 0)
            key = lax.shift_right_logical(v, 23) & 511
            for l in range(L):
                @pl.when(valid[l] == 1)
                def _():
                    k = key[l]
                    hist_s[k] = hist_s[k] + 1
            return 0

        lax.fori_loop(0, nch, hist_body, 0)

        def grp_body(k, ng):
            c = hist_s[k]

            @pl.when(c > 0)
            def _():
                scalar_write(gkey_v, ng, k)
                scalar_write(gcnt_v, ng, c)

            return ng + jnp.where(c > 0, 1, 0)

        ng = lax.fori_loop(0, 512, grp_body, 0)

        def issue_tile(gi, parity):
            k = scalar_read(gkey_v, gi)
            is_rel = lax.shift_right_logical(k, 8)
            glob = (k & 255) + g0
            off = pl.multiple_of(glob * 128, 128)
            dstbuf = tbuf.at[pl.ds(parity * 64, 64), :]

            @pl.when((is_rel == 0) & (glob < NTILE - 1))
            def _():
                pltpu.async_copy(entT.at[:, pl.ds(off, 128)], dstbuf, sem_t)

            @pl.when((is_rel == 1) & (glob < NTILE - 1))
            def _():
                pltpu.async_copy(relT.at[:, pl.ds(off, 128)], dstbuf, sem_t)

            @pl.when((is_rel == 0) & (glob == NTILE - 1))
            def _():
                pltpu.async_copy(entTailT.at[:, :], dstbuf, sem_t)

            @pl.when((is_rel == 1) & (glob == NTILE - 1))
            def _():
                pltpu.async_copy(relTailT.at[:, :], dstbuf, sem_t)

        @pl.when(ng > 0)
        def _():
            issue_tile(0, 0)

        def grp_loop(gi, s):
            lp, hc = s
            pltpu.make_async_copy(
                entT.at[:, pl.ds(0, 128)], tbuf.at[pl.ds(0, 64), :], sem_t
            ).wait()

            @pl.when(gi + 1 < ng)
            def _():
                issue_tile(gi + 1, (gi + 1) & 1)

            parity = gi & 1
            cnt = scalar_read(gcnt_v, gi)
            end = lp + cnt
            c0 = lax.shift_right_logical(lp, 4)
            ncg = lax.shift_right_logical(end - 1, 4) - c0 + 1

            def chunk_body(ci, hc2):
                cc = c0 + ci
                v = lst[pl.ds(cc * L, L)]
                gidx = cc * L + lane
                mv = jnp.where((gidx >= lp) & (gidx < end), 1, 0)
                elow = lax.shift_right_logical(v, 16) & 127
                dstp = v & 0xFFFF
                hits = jnp.cumsum(mv) - mv
                for l in range(L):
                    @pl.when(mv[l] == 1)
                    def _():
                        slot = (hc2 + hits[l]) & 127
                        col = jnp.full((L,), elow[l], dtype=_i32)
                        for kk in range(4):
                            rows = parity * 64 + kk * L + lane
                            seg = plsc.load_gather(tbuf, [rows, col])
                            res_v[slot, pl.ds(kk * L, L)] = seg
                        pltpu.async_copy(
                            res_v.at[pl.ds(slot, 1), :],
                            g_hbm.at[pl.ds(dstp[l], 1), :], sem_o)

                        @pl.when(((hc2 + hits[l]) & 127) == 127)
                        def _():
                            pltpu.make_async_copy(
                                g_hbm.at[pl.ds(0, 128), :], res_v, sem_o
                            ).wait()

                nmv = plsc.all_reduce_population_count(mv == 1)[0]
                return hc2 + nmv

            hc = lax.fori_loop(0, ncg, chunk_body, hc)
            return end, hc

        _, hcf = lax.fori_loop(0, ng, grp_loop, (0, 0))

        rem = hcf & 127

        @pl.when(rem != 0)
        def _():
            def dummy_body(i, _):
                pltpu.async_copy(
                    g_hbm.at[pl.ds(0, 1), :], trash_v.at[pl.ds(0, 1), :], sem_o)
                return 0

            lax.fori_loop(0, 128 - rem, dummy_body, 0)
            pltpu.make_async_copy(
                g_hbm.at[pl.ds(0, 128), :], res_v, sem_o).wait()

        return c_next, 0

    lax.while_loop(lambda s: s[0] < NCH, round_body, (0, 0))


BPW = B // NW
CHUNK = 128


@functools.partial(
    pl.kernel,
    out_type=jax.ShapeDtypeStruct((B,), jnp.float32),
    mesh=MESH,
    scratch_types=[
        pltpu.VMEM((CHUNK, D), jnp.float32),
        pltpu.VMEM((CHUNK, D), jnp.float32),
        pltpu.VMEM((CHUNK, D), jnp.float32),
        pltpu.VMEM((BPW,), jnp.float32),
        pltpu.SemaphoreType.DMA,
    ],
    compiler_params=CP,
)
def _score_sc(g_hbm, out_hbm, hbuf, rbuf, tbuf, out_v, sem):
    wid = lax.axis_index("s") * NC + lax.axis_index("c")
    base = wid * BPW
    for c in range(BPW // CHUNK):
        cps = [
            pltpu.async_copy(g_hbm.at[pl.ds(0 * B + base + c * CHUNK, CHUNK), :], hbuf, sem),
            pltpu.async_copy(g_hbm.at[pl.ds(1 * B + base + c * CHUNK, CHUNK), :], rbuf, sem),
            pltpu.async_copy(g_hbm.at[pl.ds(2 * B + base + c * CHUNK, CHUNK), :], tbuf, sem),
        ]
        for cp in cps:
            cp.wait()
        for g in range(CHUNK // L):
            rows = g * L + lax.iota(_i32, L)

            def body(j, acc):
                cols = jnp.full((L,), j, dtype=_i32)
                h = plsc.load_gather(hbuf, [rows, cols])
                r = plsc.load_gather(rbuf, [rows, cols])
                t = plsc.load_gather(tbuf, [rows, cols])
                return acc + jnp.abs(h + r - t)

            dist = lax.fori_loop(0, D, body, jnp.zeros((L,), jnp.float32))
            out_v[pl.ds(c * CHUNK + g * L, L)] = 1.0 / (1.0 + jnp.exp(-dist))

    pltpu.sync_copy(out_v, out_hbm.at[pl.ds(base, BPW)])


def kernel(triplets, ent_embedding, rel_embedding):
    trip = triplets.reshape(F)
    entT = ent_embedding.T
    relT = rel_embedding.T
    z = jnp.zeros((64, D), jnp.float32)
    entTailT = jnp.concatenate(
        [ent_embedding[TAIL0:TAIL0 + 64].T, z], axis=1)
    relTailT = jnp.concatenate(
        [rel_embedding[TAIL0:TAIL0 + 64].T, z], axis=1)
    g = _gather_sc(trip, entT, relT, entTailT, relTailT)
    return _score_sc(g)
